# baseline (speedup 1.0000x reference)
import functools

import jax
import jax.numpy as jnp
from jax import lax
from jax.experimental import pallas as pl
from jax.experimental.pallas import tpu as pltpu
from jax.experimental.pallas import tpu_sc as plsc

N = 10000
E = 320000
D = 128
R = 3
C = 8
HQ = 32
NR = 10240
SNP = R * NR
CHUNK = 128
BLK = 16
NBLK = 10
EDGES_PER_TILE = CHUNK * BLK * NBLK
EP = 16 * EDGES_PER_TILE
NPADE = EP - E
NB = 8
LA = 4
BE = BLK * CHUNK
RM = 4096
_SC_PARAMS = pltpu.CompilerParams(use_tc_tiling_on_sc=False)

ROWS_PER_TILE = SNP // 16


def _sc_edge_aggregate(srcp, dstp, typp, xs0, xs1):
    mesh = plsc.VectorSubcoreMesh(core_axis_name="c", subcore_axis_name="s")
    out_type = (jax.ShapeDtypeStruct((2, SNP, HQ), jnp.float32),
                jax.ShapeDtypeStruct((2, SNP, HQ), jnp.float32),
                jax.ShapeDtypeStruct((SNP,), jnp.float32))
    scratch = [
        pltpu.VMEM_SHARED((SNP, HQ), jnp.float32),
        pltpu.VMEM((NB, CHUNK, HQ), jnp.float32),
        pltpu.VMEM((2, 3, BE), jnp.int32),
        pltpu.VMEM((2, 2, BLK, CHUNK), jnp.int32),
        pltpu.SemaphoreType.DMA,
        pltpu.SemaphoreType.DMA,
        pltpu.SemaphoreType.DMA,
        pltpu.VMEM_SHARED((SNP,), jnp.float32),
        pltpu.VMEM((CHUNK,), jnp.float32),
        pltpu.VMEM((ROWS_PER_TILE,), jnp.float32),
        pltpu.SemaphoreType.DMA,
    ]

    @functools.partial(
        pl.kernel,
        out_type=out_type,
        mesh=mesh,
        compiler_params=_SC_PARAMS,
        scratch_types=scratch,
    )
    def k(src_h, dst_h, typ_h, xs0_h, xs1_h, s0_out, s1_out, cnt_out,
          s_sp, rows_v, idxb, sgb, gsem, ssem, isem,
          cnt_sp, onesv, zcv, csem):
        cid = lax.axis_index("c")
        tid = lax.axis_index("s")

        zf32 = jnp.zeros((16,), jnp.float32)
        of32 = jnp.ones((16,), jnp.float32)

        def fill_body(i, _):
            for j in range(HQ // 16):
                rows_v[0, i, pl.ds(j * 16, 16)] = zf32
            return 0
        lax.fori_loop(0, CHUNK, fill_body, 0)
        for j in range(CHUNK // 16):
            onesv[pl.ds(j * 16, 16)] = of32

        def zc_body(i, _):
            zcv[pl.ds(i * 16, 16)] = zf32
            return 0
        lax.fori_loop(0, ROWS_PER_TILE // 16, zc_body, 0)

        row0 = tid * ROWS_PER_TILE

        def zero_s():
            def zero_body(kk, _):
                pltpu.sync_copy(rows_v.at[0],
                                s_sp.at[pl.ds(row0 + kk * CHUNK, CHUNK)])
                return 0
            lax.fori_loop(0, ROWS_PER_TILE // CHUNK, zero_body, 0)

        zero_s()
        pltpu.sync_copy(zcv, cnt_sp.at[pl.ds(row0, ROWS_PER_TILE)])

        plsc.subcore_barrier()

        ebase = tid * EDGES_PER_TILE
        cn = cid * N

        def load_idx(g, buf):
            off = ebase + g * BE
            return [pltpu.async_copy(src_h.at[pl.ds(off, BE)],
                                     idxb.at[buf, 0], isem),
                    pltpu.async_copy(dst_h.at[pl.ds(off, BE)],
                                     idxb.at[buf, 1], isem),
                    pltpu.async_copy(typ_h.at[pl.ds(off, BE)],
                                     idxb.at[buf, 2], isem)]

        def compute_sg(buf):
            for b in range(BLK):
                for jq in range(CHUNK // 16):
                    sl = pl.ds(b * CHUNK + jq * 16, 16)
                    col = pl.ds(jq * 16, 16)
                    sgb[buf, 0, b, col] = (
                        idxb[buf, 2, sl] * NR + idxb[buf, 1, sl])
                    sgb[buf, 1, b, col] = idxb[buf, 0, sl] + cn

        def run_phase(xs_h, s_out, with_cnt):
            for d in load_idx(0, 0):
                d.wait()
            compute_sg(0)

            def block_body(g, _):
                buf = g % 2
                nxt = (g + 1) % 2
                idescs = load_idx(lax.rem(g + 1, NBLK), nxt)

                gd = [None] * BLK
                sd = [None] * BLK
                cd = [None] * BLK
                for b0 in range(LA):
                    gd[b0] = pltpu.async_copy(xs_h.at[sgb.at[buf, 1, b0]],
                                              rows_v.at[b0], gsem)
                for b in range(BLK):
                    gd[b].wait()
                    sd[b] = pltpu.async_copy(rows_v.at[b % NB],
                                             s_sp.at[sgb.at[buf, 0, b]],
                                             ssem, add=True)
                    if with_cnt:
                        cd[b] = pltpu.async_copy(onesv,
                                                 cnt_sp.at[sgb.at[buf, 0, b]],
                                                 csem, add=True)
                    if b >= LA:
                        sd[b - LA].wait()
                        if with_cnt:
                            cd[b - LA].wait()
                    if b + LA < BLK:
                        gd[b + LA] = pltpu.async_copy(
                            xs_h.at[sgb.at[buf, 1, b + LA]],
                            rows_v.at[(b + LA) % NB], gsem)
                for b in range(BLK - LA, BLK):
                    sd[b].wait()
                    if with_cnt:
                        cd[b].wait()

                for d in idescs:
                    d.wait()
                compute_sg(nxt)
                return 0
            lax.fori_loop(0, NBLK, block_body, 0)

            plsc.subcore_barrier()

            def wb_body(kk, _):
                r = row0 + kk * CHUNK
                pltpu.sync_copy(s_sp.at[pl.ds(r, CHUNK)],
                                s_out.at[cid, pl.ds(r, CHUNK)])
                return 0
            lax.fori_loop(0, ROWS_PER_TILE // CHUNK, wb_body, 0)

        run_phase(xs0_h, s0_out, True)

        @pl.when(cid == 0)
        def _():
            pltpu.sync_copy(cnt_sp.at[pl.ds(row0, ROWS_PER_TILE)],
                            cnt_out.at[pl.ds(row0, ROWS_PER_TILE)])

        lax.fori_loop(0, CHUNK, fill_body, 0)
        zero_s()
        plsc.subcore_barrier()

        run_phase(xs1_h, s1_out, False)

    return k(srcp, dstp, typp, xs0, xs1)


def _tc_dense(x_pad, s0p, s1p, cbp, W_rel, W_self, b_enc2, W_cls, b_cls2):
    BN = 2048
    BP = BN // 4
    nblk = NR // BN
    PRB = NR // (4 * BP)

    def body(*refs):
        x_ref = refs[0]
        q_refs = refs[1:13]
        c_refs = refs[13:16]
        wrel_ref, wself_ref, benc_ref, wcls_ref, bcls_ref, out_ref = refs[16:]
        acc = jnp.dot(x_ref[...], wself_ref[...],
                      preferred_element_type=jnp.float32)
        for r in range(R):
            inv = 1.0 / jnp.maximum(c_refs[r][...], 1.0)
            for q in range(4):
                c, p = q // 2, q % 2
                h0 = 64 * c + 32 * p
                p4 = q_refs[q * R + r][...][0] * inv
                w32 = wrel_ref[r, h0:h0 + 32, :]
                ak = [jnp.dot(p4[:, 32 * k:32 * k + 32], w32,
                              preferred_element_type=jnp.float32)
                      for k in range(4)]
                a4 = jnp.stack(ak, axis=1)
                acc = acc + a4.reshape(BN, D)
        h = jnp.maximum(acc + benc_ref[...], 0.0)
        out_ref[...] = (jnp.dot(h, wcls_ref[...],
                                preferred_element_type=jnp.float32)
                        + bcls_ref[...])

    in_specs = [pl.BlockSpec((BN, D), lambda i: (i, 0))]
    s_args = []
    for q in range(4):
        c = q // 2
        for r in range(R):
            in_specs.append(pl.BlockSpec(
                (1, BP, D), lambda i, c=c, r=r: (c, PRB * r + i, 0)))
            s_args.append(s0p if q % 2 == 0 else s1p)
    for r in range(R):
        in_specs.append(pl.BlockSpec(
            (BP, D), lambda i, r=r: (PRB * r + i, 0)))
    in_specs += [
        pl.BlockSpec((R, D, D), lambda i: (0, 0, 0)),
        pl.BlockSpec((D, D), lambda i: (0, 0)),
        pl.BlockSpec((1, D), lambda i: (0, 0)),
        pl.BlockSpec((D, C), lambda i: (0, 0)),
        pl.BlockSpec((1, C), lambda i: (0, 0)),
    ]
    return pl.pallas_call(
        body,
        grid=(nblk,),
        in_specs=in_specs,
        out_specs=pl.BlockSpec((BN, C), lambda i: (i, 0)),
        out_shape=jax.ShapeDtypeStruct((NR, C), jnp.float32),
    )(x_pad, *s_args, cbp, cbp, cbp,
      W_rel, W_self, b_enc2, W_cls, b_cls2)


def _sc_review_gather(logits, rmp):
    mesh = plsc.VectorSubcoreMesh(core_axis_name="c", subcore_axis_name="s")
    per_w = RM // 32

    @functools.partial(
        pl.kernel,
        out_type=jax.ShapeDtypeStruct((RM, C), jnp.float32),
        mesh=mesh,
        compiler_params=_SC_PARAMS,
        scratch_types=[
            pltpu.VMEM((1, per_w), jnp.int32),
            pltpu.VMEM((per_w, C), jnp.float32),
        ],
    )
    def k(lg_h, rm_h, out_h, idxv, rows_v):
        cid = lax.axis_index("c")
        tid = lax.axis_index("s")
        wid = tid * 2 + cid
        base = wid * per_w
        pltpu.sync_copy(rm_h.at[pl.ds(base, per_w)], idxv.at[0])
        pltpu.sync_copy(lg_h.at[idxv.at[0]], rows_v)
        pltpu.sync_copy(rows_v, out_h.at[pl.ds(base, per_w)])

    return k(logits, rmp)


def kernel(x, edge_index, edge_type, movie_map, user_map, review_map,
           W_rel, W_self, b_enc, W_cls, b_cls):
    src = edge_index[0]
    dst = edge_index[1]
    ar = jnp.arange(NPADE, dtype=jnp.int32)
    srcp = jnp.concatenate([src, (ar * 37) % N])
    dstp = jnp.concatenate([dst, N + ar % (NR - N)])
    typp = jnp.concatenate([edge_type, ar % R])
    xs0 = jnp.concatenate([x[:, 0:32], x[:, 64:96]], axis=0)
    xs1 = jnp.concatenate([x[:, 32:64], x[:, 96:128]], axis=0)

    s0, s1, cnt = _sc_edge_aggregate(srcp, dstp, typp, xs0, xs1)

    cbp = jnp.broadcast_to(cnt.reshape(SNP // 4, 4, 1),
                           (SNP // 4, 4, HQ)).reshape(SNP // 4, D)
    s0p = s0.reshape(2, SNP // 4, D)
    s1p = s1.reshape(2, SNP // 4, D)
    x_pad = jnp.pad(x, ((0, NR - N), (0, 0)))
    logits = _tc_dense(x_pad, s0p, s1p, cbp, W_rel, W_self,
                       b_enc.reshape(1, D), W_cls, b_cls.reshape(1, C))

    rmp = jnp.concatenate(
        [review_map, jnp.arange(RM - 4000, dtype=jnp.int32)])
    out = _sc_review_gather(logits, rmp)
    return out[:4000]


# speedup vs baseline: 1.0722x; 1.0722x over previous
import functools

import jax
import jax.numpy as jnp
from jax import lax
from jax.experimental import pallas as pl
from jax.experimental.pallas import tpu as pltpu
from jax.experimental.pallas import tpu_sc as plsc

N = 10000
E = 320000
D = 128
R = 3
C = 8
HQ = 32
NR = 10240
SNP = R * NR
CHUNK = 128
BLK = 16
NBLK = 10
EDGES_PER_TILE = CHUNK * BLK * NBLK
EP = 16 * EDGES_PER_TILE
NPADE = EP - E
NB = 8
LA = 4
BE = BLK * CHUNK
RM = 4096
_SC_PARAMS = pltpu.CompilerParams(use_tc_tiling_on_sc=False)

ROWS_PER_TILE = SNP // 16


def _sc_edge_aggregate(srcp, dstp, typp, xs, with_cnt):
    mesh = plsc.VectorSubcoreMesh(core_axis_name="c", subcore_axis_name="s")
    out_type = [jax.ShapeDtypeStruct((2, SNP, HQ), jnp.float32)]
    scratch = [
        pltpu.VMEM_SHARED((SNP, HQ), jnp.float32),
        pltpu.VMEM((NB, CHUNK, HQ), jnp.float32),
        pltpu.VMEM((2, 3, BE), jnp.int32),
        pltpu.VMEM((2, 2, BLK, CHUNK), jnp.int32),
        pltpu.SemaphoreType.DMA,
        pltpu.SemaphoreType.DMA,
        pltpu.SemaphoreType.DMA,
    ]
    if with_cnt:
        out_type.append(jax.ShapeDtypeStruct((SNP,), jnp.float32))
        scratch += [
            pltpu.VMEM_SHARED((SNP,), jnp.float32),
            pltpu.VMEM((CHUNK,), jnp.float32),
            pltpu.VMEM((ROWS_PER_TILE,), jnp.float32),
            pltpu.SemaphoreType.DMA,
        ]

    @functools.partial(
        pl.kernel,
        out_type=tuple(out_type) if with_cnt else out_type[0],
        mesh=mesh,
        compiler_params=_SC_PARAMS,
        scratch_types=scratch,
    )
    def k(src_h, dst_h, typ_h, xs_h, s_out, *rest):
        if with_cnt:
            (cnt_out, s_sp, rows_v, idxb, sgb, gsem, ssem, isem,
             cnt_sp, onesv, zcv, csem) = rest
        else:
            s_sp, rows_v, idxb, sgb, gsem, ssem, isem = rest
        cid = lax.axis_index("c")
        tid = lax.axis_index("s")

        zf32 = jnp.zeros((16,), jnp.float32)
        of32 = jnp.ones((16,), jnp.float32)

        def fill_body(i, _):
            for j in range(HQ // 16):
                rows_v[0, i, pl.ds(j * 16, 16)] = zf32
            return 0
        lax.fori_loop(0, CHUNK, fill_body, 0)
        if with_cnt:
            for j in range(CHUNK // 16):
                onesv[pl.ds(j * 16, 16)] = of32

            def zc_body(i, _):
                zcv[pl.ds(i * 16, 16)] = zf32
                return 0
            lax.fori_loop(0, ROWS_PER_TILE // 16, zc_body, 0)

        row0 = tid * ROWS_PER_TILE

        def zero_body(kk, _):
            pltpu.sync_copy(rows_v.at[0],
                            s_sp.at[pl.ds(row0 + kk * CHUNK, CHUNK)])
            return 0
        lax.fori_loop(0, ROWS_PER_TILE // CHUNK, zero_body, 0)
        if with_cnt:
            pltpu.sync_copy(zcv, cnt_sp.at[pl.ds(row0, ROWS_PER_TILE)])

        plsc.subcore_barrier()

        ebase = tid * EDGES_PER_TILE
        cn = cid * N

        def load_idx(g, buf):
            off = ebase + g * BE
            return [pltpu.async_copy(src_h.at[pl.ds(off, BE)],
                                     idxb.at[buf, 0], isem),
                    pltpu.async_copy(dst_h.at[pl.ds(off, BE)],
                                     idxb.at[buf, 1], isem),
                    pltpu.async_copy(typ_h.at[pl.ds(off, BE)],
                                     idxb.at[buf, 2], isem)]

        def compute_sg(buf):
            for b in range(BLK):
                for jq in range(CHUNK // 16):
                    sl = pl.ds(b * CHUNK + jq * 16, 16)
                    col = pl.ds(jq * 16, 16)
                    sgb[buf, 0, b, col] = (
                        idxb[buf, 2, sl] * NR + idxb[buf, 1, sl])
                    sgb[buf, 1, b, col] = idxb[buf, 0, sl] + cn

        if True:
            for d in load_idx(0, 0):
                d.wait()
            compute_sg(0)

            def block_body(g, _):
                buf = g % 2
                nxt = (g + 1) % 2
                idescs = load_idx(lax.rem(g + 1, NBLK), nxt)

                gd = [None] * BLK
                sd = [None] * BLK
                cd = [None] * BLK
                for b0 in range(LA):
                    gd[b0] = pltpu.async_copy(xs_h.at[sgb.at[buf, 1, b0]],
                                              rows_v.at[b0], gsem)
                for b in range(BLK):
                    gd[b].wait()
                    sd[b] = pltpu.async_copy(rows_v.at[b % NB],
                                             s_sp.at[sgb.at[buf, 0, b]],
                                             ssem, add=True)
                    if with_cnt:
                        cd[b] = pltpu.async_copy(onesv,
                                                 cnt_sp.at[sgb.at[buf, 0, b]],
                                                 csem, add=True)
                    if b >= LA:
                        sd[b - LA].wait()
                        if with_cnt:
                            cd[b - LA].wait()
                    if b + LA < BLK:
                        gd[b + LA] = pltpu.async_copy(
                            xs_h.at[sgb.at[buf, 1, b + LA]],
                            rows_v.at[(b + LA) % NB], gsem)
                for b in range(BLK - LA, BLK):
                    sd[b].wait()
                    if with_cnt:
                        cd[b].wait()

                for d in idescs:
                    d.wait()
                compute_sg(nxt)
                return 0
            lax.fori_loop(0, NBLK, block_body, 0)

            plsc.subcore_barrier()

            def wb_body(kk, _):
                r = row0 + kk * CHUNK
                pltpu.sync_copy(s_sp.at[pl.ds(r, CHUNK)],
                                s_out.at[cid, pl.ds(r, CHUNK)])
                return 0
            lax.fori_loop(0, ROWS_PER_TILE // CHUNK, wb_body, 0)

        if with_cnt:
            @pl.when(cid == 0)
            def _():
                pltpu.sync_copy(cnt_sp.at[pl.ds(row0, ROWS_PER_TILE)],
                                cnt_out.at[pl.ds(row0, ROWS_PER_TILE)])

    return k(srcp, dstp, typp, xs)


def _tc_dense(x_pad, s0p, s1p, cbp, W_rel, W_self, b_enc2, W_cls, b_cls2):
    BN = 2048
    BP = BN // 4
    nblk = NR // BN
    PRB = NR // (4 * BP)

    def body(*refs):
        x_ref = refs[0]
        q_refs = refs[1:13]
        c_refs = refs[13:16]
        wrel_ref, wself_ref, benc_ref, wcls_ref, bcls_ref, out_ref = refs[16:]
        acc = jnp.dot(x_ref[...], wself_ref[...],
                      preferred_element_type=jnp.float32)
        for r in range(R):
            inv = 1.0 / jnp.maximum(c_refs[r][...], 1.0)
            for q in range(4):
                c, p = q // 2, q % 2
                h0 = 64 * c + 32 * p
                p4 = q_refs[q * R + r][...][0] * inv
                w32 = wrel_ref[r, h0:h0 + 32, :]
                ak = [jnp.dot(p4[:, 32 * k:32 * k + 32], w32,
                              preferred_element_type=jnp.float32)
                      for k in range(4)]
                a4 = jnp.stack(ak, axis=1)
                acc = acc + a4.reshape(BN, D)
        h = jnp.maximum(acc + benc_ref[...], 0.0)
        out_ref[...] = (jnp.dot(h, wcls_ref[...],
                                preferred_element_type=jnp.float32)
                        + bcls_ref[...])

    in_specs = [pl.BlockSpec((BN, D), lambda i: (i, 0))]
    s_args = []
    for q in range(4):
        c = q // 2
        for r in range(R):
            in_specs.append(pl.BlockSpec(
                (1, BP, D), lambda i, c=c, r=r: (c, PRB * r + i, 0)))
            s_args.append(s0p if q % 2 == 0 else s1p)
    for r in range(R):
        in_specs.append(pl.BlockSpec(
            (BP, D), lambda i, r=r: (PRB * r + i, 0)))
    in_specs += [
        pl.BlockSpec((R, D, D), lambda i: (0, 0, 0)),
        pl.BlockSpec((D, D), lambda i: (0, 0)),
        pl.BlockSpec((1, D), lambda i: (0, 0)),
        pl.BlockSpec((D, C), lambda i: (0, 0)),
        pl.BlockSpec((1, C), lambda i: (0, 0)),
    ]
    return pl.pallas_call(
        body,
        grid=(nblk,),
        in_specs=in_specs,
        out_specs=pl.BlockSpec((BN, C), lambda i: (i, 0)),
        out_shape=jax.ShapeDtypeStruct((NR, C), jnp.float32),
    )(x_pad, *s_args, cbp, cbp, cbp,
      W_rel, W_self, b_enc2, W_cls, b_cls2)


def _sc_review_gather(logits, rmp):
    mesh = plsc.VectorSubcoreMesh(core_axis_name="c", subcore_axis_name="s")
    per_w = RM // 32

    @functools.partial(
        pl.kernel,
        out_type=jax.ShapeDtypeStruct((RM, C), jnp.float32),
        mesh=mesh,
        compiler_params=_SC_PARAMS,
        scratch_types=[
            pltpu.VMEM((1, per_w), jnp.int32),
            pltpu.VMEM((per_w, C), jnp.float32),
        ],
    )
    def k(lg_h, rm_h, out_h, idxv, rows_v):
        cid = lax.axis_index("c")
        tid = lax.axis_index("s")
        wid = tid * 2 + cid
        base = wid * per_w
        pltpu.sync_copy(rm_h.at[pl.ds(base, per_w)], idxv.at[0])
        pltpu.sync_copy(lg_h.at[idxv.at[0]], rows_v)
        pltpu.sync_copy(rows_v, out_h.at[pl.ds(base, per_w)])

    return k(logits, rmp)


def kernel(x, edge_index, edge_type, movie_map, user_map, review_map,
           W_rel, W_self, b_enc, W_cls, b_cls):
    src = edge_index[0]
    dst = edge_index[1]
    ar = jnp.arange(NPADE, dtype=jnp.int32)
    srcp = jnp.concatenate([src, (ar * 37) % N])
    dstp = jnp.concatenate([dst, N + ar % (NR - N)])
    typp = jnp.concatenate([edge_type, ar % R])
    xs0 = jnp.concatenate([x[:, 0:32], x[:, 64:96]], axis=0)
    xs1 = jnp.concatenate([x[:, 32:64], x[:, 96:128]], axis=0)

    s0, cnt = _sc_edge_aggregate(srcp, dstp, typp, xs0, True)
    s1 = _sc_edge_aggregate(srcp, dstp, typp, xs1, False)

    cbp = jnp.broadcast_to(cnt.reshape(SNP // 4, 4, 1),
                           (SNP // 4, 4, HQ)).reshape(SNP // 4, D)
    s0p = s0.reshape(2, SNP // 4, D)
    s1p = s1.reshape(2, SNP // 4, D)
    x_pad = jnp.pad(x, ((0, NR - N), (0, 0)))
    logits = _tc_dense(x_pad, s0p, s1p, cbp, W_rel, W_self,
                       b_enc.reshape(1, D), W_cls, b_cls.reshape(1, C))

    rmp = jnp.concatenate(
        [review_map, jnp.arange(RM - 4000, dtype=jnp.int32)])
    out = _sc_review_gather(logits, rmp)
    return out[:4000]


# speedup vs baseline: 1.0863x; 1.0132x over previous
import functools

import jax
import jax.numpy as jnp
from jax import lax
from jax.experimental import pallas as pl
from jax.experimental.pallas import tpu as pltpu
from jax.experimental.pallas import tpu_sc as plsc

N = 10000
E = 320000
D = 128
R = 3
C = 8
HQ = 32
NR = 10240
SNP = R * NR
CHUNK = 128
BLK = 16
NBLK = 10
EDGES_PER_TILE = CHUNK * BLK * NBLK
EP = 16 * EDGES_PER_TILE
NPADE = EP - E
NB = 10
LA = 5
BE = BLK * CHUNK
RM = 4096
_SC_PARAMS = pltpu.CompilerParams(use_tc_tiling_on_sc=False)

ROWS_PER_TILE = SNP // 16


def _sc_edge_aggregate(srcp, dstp, typp, xs, with_cnt):
    mesh = plsc.VectorSubcoreMesh(core_axis_name="c", subcore_axis_name="s")
    out_type = [jax.ShapeDtypeStruct((2, SNP, HQ), jnp.float32)]
    scratch = [
        pltpu.VMEM_SHARED((SNP, HQ), jnp.float32),
        pltpu.VMEM((NB, CHUNK, HQ), jnp.float32),
        pltpu.VMEM((2, 3, BE), jnp.int32),
        pltpu.VMEM((2, 2, BLK, CHUNK), jnp.int32),
        pltpu.SemaphoreType.DMA,
        pltpu.SemaphoreType.DMA,
        pltpu.SemaphoreType.DMA,
    ]
    if with_cnt:
        out_type.append(jax.ShapeDtypeStruct((SNP,), jnp.float32))
        scratch += [
            pltpu.VMEM_SHARED((SNP,), jnp.float32),
            pltpu.VMEM((CHUNK,), jnp.float32),
            pltpu.VMEM((ROWS_PER_TILE,), jnp.float32),
            pltpu.SemaphoreType.DMA,
        ]

    @functools.partial(
        pl.kernel,
        out_type=tuple(out_type) if with_cnt else out_type[0],
        mesh=mesh,
        compiler_params=_SC_PARAMS,
        scratch_types=scratch,
    )
    def k(src_h, dst_h, typ_h, xs_h, s_out, *rest):
        if with_cnt:
            (cnt_out, s_sp, rows_v, idxb, sgb, gsem, ssem, isem,
             cnt_sp, onesv, zcv, csem) = rest
        else:
            s_sp, rows_v, idxb, sgb, gsem, ssem, isem = rest
        cid = lax.axis_index("c")
        tid = lax.axis_index("s")

        zf32 = jnp.zeros((16,), jnp.float32)
        of32 = jnp.ones((16,), jnp.float32)

        def fill_body(i, _):
            for j in range(HQ // 16):
                rows_v[0, i, pl.ds(j * 16, 16)] = zf32
            return 0
        lax.fori_loop(0, CHUNK, fill_body, 0)
        if with_cnt:
            for j in range(CHUNK // 16):
                onesv[pl.ds(j * 16, 16)] = of32

            def zc_body(i, _):
                zcv[pl.ds(i * 16, 16)] = zf32
                return 0
            lax.fori_loop(0, ROWS_PER_TILE // 16, zc_body, 0)

        row0 = tid * ROWS_PER_TILE

        def zero_body(kk, _):
            pltpu.sync_copy(rows_v.at[0],
                            s_sp.at[pl.ds(row0 + kk * CHUNK, CHUNK)])
            return 0
        lax.fori_loop(0, ROWS_PER_TILE // CHUNK, zero_body, 0)
        if with_cnt:
            pltpu.sync_copy(zcv, cnt_sp.at[pl.ds(row0, ROWS_PER_TILE)])

        plsc.subcore_barrier()

        ebase = tid * EDGES_PER_TILE
        cn = cid * N

        def load_idx(g, buf):
            off = ebase + g * BE
            return [pltpu.async_copy(src_h.at[pl.ds(off, BE)],
                                     idxb.at[buf, 0], isem),
                    pltpu.async_copy(dst_h.at[pl.ds(off, BE)],
                                     idxb.at[buf, 1], isem),
                    pltpu.async_copy(typ_h.at[pl.ds(off, BE)],
                                     idxb.at[buf, 2], isem)]

        def compute_sg(buf):
            for b in range(BLK):
                for jq in range(CHUNK // 16):
                    sl = pl.ds(b * CHUNK + jq * 16, 16)
                    col = pl.ds(jq * 16, 16)
                    sgb[buf, 0, b, col] = (
                        idxb[buf, 2, sl] * NR + idxb[buf, 1, sl])
                    sgb[buf, 1, b, col] = idxb[buf, 0, sl] + cn

        if True:
            for d in load_idx(0, 0):
                d.wait()
            compute_sg(0)

            def block_body(g, _):
                buf = g % 2
                nxt = (g + 1) % 2
                idescs = load_idx(lax.rem(g + 1, NBLK), nxt)

                gd = [None] * BLK
                sd = [None] * BLK
                cd = [None] * BLK
                for b0 in range(LA):
                    gd[b0] = pltpu.async_copy(xs_h.at[sgb.at[buf, 1, b0]],
                                              rows_v.at[b0], gsem)
                for b in range(BLK):
                    gd[b].wait()
                    sd[b] = pltpu.async_copy(rows_v.at[b % NB],
                                             s_sp.at[sgb.at[buf, 0, b]],
                                             ssem, add=True)
                    if with_cnt:
                        cd[b] = pltpu.async_copy(onesv,
                                                 cnt_sp.at[sgb.at[buf, 0, b]],
                                                 csem, add=True)
                    if b >= LA:
                        sd[b - LA].wait()
                        if with_cnt:
                            cd[b - LA].wait()
                    if b + LA < BLK:
                        gd[b + LA] = pltpu.async_copy(
                            xs_h.at[sgb.at[buf, 1, b + LA]],
                            rows_v.at[(b + LA) % NB], gsem)
                for b in range(BLK - LA, BLK):
                    sd[b].wait()
                    if with_cnt:
                        cd[b].wait()

                for d in idescs:
                    d.wait()
                compute_sg(nxt)
                return 0
            lax.fori_loop(0, NBLK, block_body, 0)

            plsc.subcore_barrier()

            def wb_body(kk, _):
                r = row0 + kk * CHUNK
                pltpu.sync_copy(s_sp.at[pl.ds(r, CHUNK)],
                                s_out.at[cid, pl.ds(r, CHUNK)])
                return 0
            lax.fori_loop(0, ROWS_PER_TILE // CHUNK, wb_body, 0)

        if with_cnt:
            @pl.when(cid == 0)
            def _():
                pltpu.sync_copy(cnt_sp.at[pl.ds(row0, ROWS_PER_TILE)],
                                cnt_out.at[pl.ds(row0, ROWS_PER_TILE)])

    return k(srcp, dstp, typp, xs)


def _tc_dense(x_pad, s0p, s1p, cbp, W_rel, W_self, b_enc2, W_cls, b_cls2):
    BN = 2048
    BP = BN // 4
    nblk = NR // BN
    PRB = NR // (4 * BP)

    def body(*refs):
        x_ref = refs[0]
        q_refs = refs[1:13]
        c_refs = refs[13:16]
        wrel_ref, wself_ref, benc_ref, wcls_ref, bcls_ref, out_ref = refs[16:]
        acc = jnp.dot(x_ref[...], wself_ref[...],
                      preferred_element_type=jnp.float32)
        for r in range(R):
            inv = 1.0 / jnp.maximum(c_refs[r][...], 1.0)
            for q in range(4):
                c, p = q // 2, q % 2
                h0 = 64 * c + 32 * p
                p4 = q_refs[q * R + r][...][0] * inv
                w32 = wrel_ref[r, h0:h0 + 32, :]
                ak = [jnp.dot(p4[:, 32 * k:32 * k + 32], w32,
                              preferred_element_type=jnp.float32)
                      for k in range(4)]
                a4 = jnp.stack(ak, axis=1)
                acc = acc + a4.reshape(BN, D)
        h = jnp.maximum(acc + benc_ref[...], 0.0)
        out_ref[...] = (jnp.dot(h, wcls_ref[...],
                                preferred_element_type=jnp.float32)
                        + bcls_ref[...])

    in_specs = [pl.BlockSpec((BN, D), lambda i: (i, 0))]
    s_args = []
    for q in range(4):
        c = q // 2
        for r in range(R):
            in_specs.append(pl.BlockSpec(
                (1, BP, D), lambda i, c=c, r=r: (c, PRB * r + i, 0)))
            s_args.append(s0p if q % 2 == 0 else s1p)
    for r in range(R):
        in_specs.append(pl.BlockSpec(
            (BP, D), lambda i, r=r: (PRB * r + i, 0)))
    in_specs += [
        pl.BlockSpec((R, D, D), lambda i: (0, 0, 0)),
        pl.BlockSpec((D, D), lambda i: (0, 0)),
        pl.BlockSpec((1, D), lambda i: (0, 0)),
        pl.BlockSpec((D, C), lambda i: (0, 0)),
        pl.BlockSpec((1, C), lambda i: (0, 0)),
    ]
    return pl.pallas_call(
        body,
        grid=(nblk,),
        in_specs=in_specs,
        out_specs=pl.BlockSpec((BN, C), lambda i: (i, 0)),
        out_shape=jax.ShapeDtypeStruct((NR, C), jnp.float32),
    )(x_pad, *s_args, cbp, cbp, cbp,
      W_rel, W_self, b_enc2, W_cls, b_cls2)


def _sc_review_gather(logits, rmp):
    mesh = plsc.VectorSubcoreMesh(core_axis_name="c", subcore_axis_name="s")
    per_w = RM // 32

    @functools.partial(
        pl.kernel,
        out_type=jax.ShapeDtypeStruct((RM, C), jnp.float32),
        mesh=mesh,
        compiler_params=_SC_PARAMS,
        scratch_types=[
            pltpu.VMEM((1, per_w), jnp.int32),
            pltpu.VMEM((per_w, C), jnp.float32),
        ],
    )
    def k(lg_h, rm_h, out_h, idxv, rows_v):
        cid = lax.axis_index("c")
        tid = lax.axis_index("s")
        wid = tid * 2 + cid
        base = wid * per_w
        pltpu.sync_copy(rm_h.at[pl.ds(base, per_w)], idxv.at[0])
        pltpu.sync_copy(lg_h.at[idxv.at[0]], rows_v)
        pltpu.sync_copy(rows_v, out_h.at[pl.ds(base, per_w)])

    return k(logits, rmp)


def kernel(x, edge_index, edge_type, movie_map, user_map, review_map,
           W_rel, W_self, b_enc, W_cls, b_cls):
    src = edge_index[0]
    dst = edge_index[1]
    ar = jnp.arange(NPADE, dtype=jnp.int32)
    srcp = jnp.concatenate([src, (ar * 37) % N])
    dstp = jnp.concatenate([dst, N + ar % (NR - N)])
    typp = jnp.concatenate([edge_type, ar % R])
    xs0 = jnp.concatenate([x[:, 0:32], x[:, 64:96]], axis=0)
    xs1 = jnp.concatenate([x[:, 32:64], x[:, 96:128]], axis=0)

    s0, cnt = _sc_edge_aggregate(srcp, dstp, typp, xs0, True)
    s1 = _sc_edge_aggregate(srcp, dstp, typp, xs1, False)

    cbp = jnp.broadcast_to(cnt.reshape(SNP // 4, 4, 1),
                           (SNP // 4, 4, HQ)).reshape(SNP // 4, D)
    s0p = s0.reshape(2, SNP // 4, D)
    s1p = s1.reshape(2, SNP // 4, D)
    x_pad = jnp.pad(x, ((0, NR - N), (0, 0)))
    logits = _tc_dense(x_pad, s0p, s1p, cbp, W_rel, W_self,
                       b_enc.reshape(1, D), W_cls, b_cls.reshape(1, C))

    rmp = jnp.concatenate(
        [review_map, jnp.arange(RM - 4000, dtype=jnp.int32)])
    out = _sc_review_gather(logits, rmp)
    return out[:4000]
